# Initial kernel scaffold; baseline (speedup 1.0000x reference)
import jax
import jax.numpy as jnp
from jax.experimental import pallas as pl


def kernel(X, W, bias):
    raise NotImplementedError("write your pallas kernel here")


# speedup vs baseline: 1.2951x; 1.2951x over previous
import functools

import jax
import jax.numpy as jnp
from jax import lax
from jax.experimental import pallas as pl
from jax.experimental.pallas import tpu as pltpu
from jax.experimental.pallas import tpu_sc as plsc

BATCH = 16384
N_FIELDS = 100
LANES = 16
NW = 32
B_PER_W = BATCH // NW
IDX_PER_W = B_PER_W * N_FIELDS

_mesh = plsc.VectorSubcoreMesh(core_axis_name="c", subcore_axis_name="s")


@functools.partial(
    pl.kernel,
    mesh=_mesh,
    out_type=jax.ShapeDtypeStruct((BATCH,), jnp.float32),
    scratch_types=[
        pltpu.VMEM((IDX_PER_W,), jnp.int32),
        pltpu.VMEM((IDX_PER_W,), jnp.float32),
        pltpu.VMEM((B_PER_W,), jnp.float32),
        pltpu.VMEM((LANES,), jnp.float32),
        pltpu.SemaphoreType.DMA,
    ],
)
def _wide_kernel(xt_hbm, w_hbm, bias_hbm, out_hbm, idx_v, vals_v, out_v, bias_v, sem):
    wid = lax.axis_index("c") * 16 + lax.axis_index("s")
    base_b = wid * B_PER_W

    pltpu.sync_copy(bias_hbm, bias_v)
    pltpu.sync_copy(xt_hbm.at[wid], idx_v)
    pltpu.async_copy(w_hbm.at[idx_v], vals_v, sem).wait()

    bias_vec = bias_v[...]

    def jbody(j, _):
        j16 = j * LANES
        acc = bias_vec
        for f in range(N_FIELDS):
            acc = acc + vals_v[pl.ds(f * B_PER_W + j16, LANES)]
        out_v[pl.ds(j16, LANES)] = acc
        return 0

    lax.fori_loop(0, B_PER_W // LANES, jbody, 0)
    pltpu.sync_copy(out_v, out_hbm.at[pl.ds(base_b, B_PER_W)])


def kernel(X, W, bias):
    xt = (
        X.astype(jnp.int32)
        .T.reshape(N_FIELDS, NW, B_PER_W)
        .swapaxes(0, 1)
        .reshape(NW, IDX_PER_W)
    )
    w_flat = W.reshape(-1)
    bias16 = jnp.broadcast_to(bias.astype(jnp.float32), (LANES,))
    out = _wide_kernel(xt, w_flat, bias16)
    return out.reshape(BATCH, 1)
